# Initial kernel scaffold; baseline (speedup 1.0000x reference)
import jax
import jax.numpy as jnp
from jax.experimental import pallas as pl


def kernel(x_real, x_fake, n_bins):
    raise NotImplementedError("write your pallas kernel here")


# speedup vs baseline: 26.8563x; 26.8563x over previous
import functools

import jax
import jax.numpy as jnp
from jax import lax
from jax.experimental import pallas as pl
from jax.experimental.pallas import tpu as pltpu
from jax.experimental.pallas import tpu_sc as plsc

N, L, D, NBINS = 4096, 64, 32, 256
NC, NS = 2, 16
NW = NC * NS
CPW = (L * D) // NW
G = CPW // 16
CHUNK = 512
NCHUNK = N // CHUNK


def _hist_pass(buf, hist, lo, hi, scale, base):
    ones = jnp.ones((16,), jnp.float32)

    def body(i, c):
        for g in range(G):
            x = buf[i, pl.ds(g * 16, 16)]
            tb = (x - lo[g]) * scale[g]
            tb = jnp.minimum(jnp.maximum(tb, 0.0), 511.0)
            idx = jnp.minimum(tb.astype(jnp.int32), NBINS - 1)
            within = (x >= lo[g]) & (x <= hi[g])
            plsc.addupdate_scatter(hist, [idx + base[g]], ones, mask=within)
        return c

    lax.fori_loop(0, CHUNK, body, 0)


def _kernel_body(xr_hbm, xf_hbm, out_hbm, buf, hist_r, hist_f, loss_v):
    wid = lax.axis_index("c") * NS + lax.axis_index("s")
    iota = lax.iota(jnp.int32, 16)
    base = [(g * 16 + iota) * NBINS for g in range(G)]

    mns = [jnp.full((16,), jnp.inf, jnp.float32) for _ in range(G)]
    mxs = [jnp.full((16,), -jnp.inf, jnp.float32) for _ in range(G)]
    for ch in range(NCHUNK):
        pltpu.sync_copy(xr_hbm.at[pl.ds(ch * CHUNK, CHUNK), wid], buf)

        def mmbody(i, carry):
            mns_c, mxs_c = carry
            new_mn, new_mx = [], []
            for g in range(G):
                x = buf[i, pl.ds(g * 16, 16)]
                new_mn.append(jnp.minimum(mns_c[g], x))
                new_mx.append(jnp.maximum(mxs_c[g], x))
            return tuple(new_mn), tuple(new_mx)

        mns, mxs = lax.fori_loop(0, CHUNK, mmbody, (tuple(mns), tuple(mxs)))
        mns, mxs = list(mns), list(mxs)

    lo, hi, scale = [], [], []
    for g in range(G):
        mn, mx = mns[g], mxs[g]
        same = jnp.abs(mx - mn) < 1e-10
        mx = jnp.where(same, mx + 1e-5, mx)
        mn = jnp.where(same, mn - 1e-5, mn)
        lo.append(mn)
        hi.append(mx)
        scale.append((1.0 / (mx - mn)) * jnp.float32(NBINS))

    zeros = jnp.zeros((16,), jnp.float32)

    def zbody(i, c):
        hist_r[pl.ds(i * 16, 16)] = zeros
        hist_f[pl.ds(i * 16, 16)] = zeros
        return c

    lax.fori_loop(0, CPW * NBINS // 16, zbody, 0)

    for ch in range(NCHUNK):
        pltpu.sync_copy(xr_hbm.at[pl.ds(ch * CHUNK, CHUNK), wid], buf)
        _hist_pass(buf, hist_r, lo, hi, scale, base)

    for ch in range(NCHUNK):
        pltpu.sync_copy(xf_hbm.at[pl.ds(ch * CHUNK, CHUNK), wid], buf)
        _hist_pass(buf, hist_f, lo, hi, scale, base)

    inv_n = jnp.float32(1.0 / N)
    for g in range(G):
        colbase = base[g]

        def fbody(b, carry):
            sa, sf = carry
            cr = plsc.load_gather(hist_r, [colbase + b])
            cf = plsc.load_gather(hist_f, [colbase + b])
            return sa + jnp.abs(cf - cr), sf + cf

        sa, sf = lax.fori_loop(0, NBINS, fbody, (zeros, zeros))
        loss_g = sa * inv_n + (jnp.float32(N) - sf) * inv_n
        bw = (hi[g] - lo[g]) * jnp.float32(1.0 / NBINS)
        c_first = lo[g] + bw * jnp.float32(0.5)
        c_last = lo[g] + bw * jnp.float32(NBINS - 0.5)
        deg = (jnp.abs(c_first) < 1e-16) & (jnp.abs(c_last) < 1e-16)
        loss_g = jnp.where(deg, jnp.float32(2.0), loss_g)
        loss_v[pl.ds(g * 16, 16)] = loss_g

    pltpu.sync_copy(loss_v, out_hbm.at[pl.ds(wid * CPW, CPW)])


@jax.jit
def _hist_loss(xr, xf):
    mesh = plsc.VectorSubcoreMesh(
        core_axis_name="c", subcore_axis_name="s", num_cores=NC, num_subcores=NS
    )
    return pl.kernel(
        _kernel_body,
        out_type=jax.ShapeDtypeStruct((L * D,), jnp.float32),
        mesh=mesh,
        compiler_params=pltpu.CompilerParams(needs_layout_passes=False),
        scratch_types=[
            pltpu.VMEM((CHUNK, CPW), jnp.float32),
            pltpu.VMEM((CPW * NBINS,), jnp.float32),
            pltpu.VMEM((CPW * NBINS,), jnp.float32),
            pltpu.VMEM((CPW,), jnp.float32),
        ],
    )(xr, xf)


def kernel(x_real, x_fake, n_bins):
    del n_bins
    xr = x_real.reshape(N, NW, CPW)
    xf = x_fake.reshape(N, NW, CPW)
    return _hist_loss(xr, xf).reshape(L, D)


# speedup vs baseline: 64.2583x; 2.3927x over previous
import jax
import jax.numpy as jnp
from jax import lax
from jax.experimental import pallas as pl
from jax.experimental.pallas import tpu as pltpu
from jax.experimental.pallas import tpu_sc as plsc

N, L, D, NBINS = 4096, 64, 32, 256
NC, NS = 2, 16
NW = NC * NS
CPW = (L * D) // NW
G = CPW // 16
CHUNK = 256
NCHUNK = N // CHUNK


def _double_buffered(src_hbm, wid, bufs, sems, consume):
    copies = [None, None]
    copies[0] = pltpu.async_copy(
        src_hbm.at[pl.ds(0, CHUNK), wid], bufs[0], sems[0]
    )
    for ch in range(NCHUNK):
        cur = ch % 2
        copies[cur].wait()
        if ch + 1 < NCHUNK:
            nxt = 1 - cur
            copies[nxt] = pltpu.async_copy(
                src_hbm.at[pl.ds((ch + 1) * CHUNK, CHUNK), wid],
                bufs[nxt],
                sems[nxt],
            )
        consume(bufs[cur])


def _kernel_body(xr_hbm, xf_hbm, out_hbm, buf0, buf1, hist_r, hist_f,
                 loss_v, sem0, sem1):
    wid = lax.axis_index("c") * NS + lax.axis_index("s")
    iota = lax.iota(jnp.int32, 16)
    base = [(g * 16 + iota) * NBINS for g in range(G)]
    bufs, sems = [buf0, buf1], [sem0, sem1]

    carry0 = (
        tuple(jnp.full((16,), jnp.inf, jnp.float32) for _ in range(G)),
        tuple(jnp.full((16,), -jnp.inf, jnp.float32) for _ in range(G)),
    )
    state = [carry0]

    def mm_consume(buf):
        def mmbody(i, carry):
            mns_c, mxs_c = carry
            new_mn, new_mx = [], []
            for g in range(G):
                x = buf[i, pl.ds(g * 16, 16)]
                new_mn.append(jnp.minimum(mns_c[g], x))
                new_mx.append(jnp.maximum(mxs_c[g], x))
            return tuple(new_mn), tuple(new_mx)

        state[0] = plsc.parallel_loop(0, CHUNK, unroll=4, carry=state[0])(
            mmbody
        )

    _double_buffered(xr_hbm, wid, bufs, sems, mm_consume)
    mns, mxs = state[0]

    lo, hi, scale = [], [], []
    for g in range(G):
        mn, mx = mns[g], mxs[g]
        same = jnp.abs(mx - mn) < 1e-10
        mx = jnp.where(same, mx + 1e-5, mx)
        mn = jnp.where(same, mn - 1e-5, mn)
        lo.append(mn)
        hi.append(mx)
        scale.append((1.0 / (mx - mn)) * jnp.float32(NBINS))

    zeros = jnp.zeros((16,), jnp.float32)

    @plsc.parallel_loop(0, CPW * NBINS // 16, unroll=4)
    def zbody(i):
        hist_r[pl.ds(i * 16, 16)] = zeros
        hist_f[pl.ds(i * 16, 16)] = zeros

    ones = jnp.ones((16,), jnp.float32)

    def hist_consume(hist):
        def consume(buf):
            @plsc.parallel_loop(0, CHUNK, unroll=2)
            def body(i):
                for g in range(G):
                    x = buf[i, pl.ds(g * 16, 16)]
                    tb = (x - lo[g]) * scale[g]
                    tb = jnp.minimum(jnp.maximum(tb, 0.0), 511.0)
                    idx = jnp.minimum(tb.astype(jnp.int32), NBINS - 1)
                    within = (x >= lo[g]) & (x <= hi[g])
                    plsc.addupdate_scatter(
                        hist, [idx + base[g]], ones, mask=within
                    )

        return consume

    _double_buffered(xr_hbm, wid, bufs, sems, hist_consume(hist_r))
    _double_buffered(xf_hbm, wid, bufs, sems, hist_consume(hist_f))

    inv_n = jnp.float32(1.0 / N)
    for g in range(G):
        colbase = base[g]

        def fbody(b, carry):
            sa, sf = carry
            cr = plsc.load_gather(hist_r, [colbase + b])
            cf = plsc.load_gather(hist_f, [colbase + b])
            return sa + jnp.abs(cf - cr), sf + cf

        sa, sf = plsc.parallel_loop(0, NBINS, unroll=4, carry=(zeros, zeros))(
            fbody
        )
        loss_g = sa * inv_n + (jnp.float32(N) - sf) * inv_n
        bw = (hi[g] - lo[g]) * jnp.float32(1.0 / NBINS)
        c_first = lo[g] + bw * jnp.float32(0.5)
        c_last = lo[g] + bw * jnp.float32(NBINS - 0.5)
        deg = (jnp.abs(c_first) < 1e-16) & (jnp.abs(c_last) < 1e-16)
        loss_g = jnp.where(deg, jnp.float32(2.0), loss_g)
        loss_v[pl.ds(g * 16, 16)] = loss_g

    pltpu.sync_copy(loss_v, out_hbm.at[pl.ds(wid * CPW, CPW)])


@jax.jit
def _hist_loss(xr, xf):
    mesh = plsc.VectorSubcoreMesh(
        core_axis_name="c", subcore_axis_name="s", num_cores=NC, num_subcores=NS
    )
    return pl.kernel(
        _kernel_body,
        out_type=jax.ShapeDtypeStruct((L * D,), jnp.float32),
        mesh=mesh,
        compiler_params=pltpu.CompilerParams(needs_layout_passes=False),
        scratch_types=[
            pltpu.VMEM((CHUNK, CPW), jnp.float32),
            pltpu.VMEM((CHUNK, CPW), jnp.float32),
            pltpu.VMEM((CPW * NBINS,), jnp.float32),
            pltpu.VMEM((CPW * NBINS,), jnp.float32),
            pltpu.VMEM((CPW,), jnp.float32),
            pltpu.SemaphoreType.DMA,
            pltpu.SemaphoreType.DMA,
        ],
    )(xr, xf)


def kernel(x_real, x_fake, n_bins):
    del n_bins
    xr = x_real.reshape(N, NW, CPW)
    xf = x_fake.reshape(N, NW, CPW)
    return _hist_loss(xr, xf).reshape(L, D)
